# Initial kernel scaffold; baseline (speedup 1.0000x reference)
import jax
import jax.numpy as jnp
from jax.experimental import pallas as pl


def kernel(x, edge_index, edge_weight, Wb1, Wc1, bc1, bias1, g1, be1, Wb2, Wc2, bc2, bias2, g2, be2, W3, b3, g3, be3, W4, b4):
    raise NotImplementedError("write your pallas kernel here")


# speedup vs baseline: 1.1180x; 1.1180x over previous
import functools

import jax
import jax.numpy as jnp
from jax import lax
from jax.experimental import pallas as pl
from jax.experimental.pallas import tpu as pltpu

H, B, A = 8, 4, 5
F = 16
BF = B * F
KA = A * B
HID = H * F
BN = 400


def _project_body(x_ref, wb_ref, wc_ref, bc_ref, bases_ref, wt_ref):
    x = x_ref[...]
    bases_ref[...] = jnp.dot(x, wb_ref[...], preferred_element_type=jnp.float32, precision=lax.Precision.HIGHEST)
    wt_ref[...] = (
        jnp.dot(x, wc_ref[...], preferred_element_type=jnp.float32, precision=lax.Precision.HIGHEST)
        + bc_ref[...]
    )


def _project(x, Wb, Wc, bc):
    n, din = x.shape
    grid = n // BN
    return pl.pallas_call(
        _project_body,
        grid=(grid,),
        in_specs=[
            pl.BlockSpec((BN, din), lambda i: (i, 0)),
            pl.BlockSpec((din, BF), lambda i: (0, 0)),
            pl.BlockSpec((din, H * KA), lambda i: (0, 0)),
            pl.BlockSpec((1, H * KA), lambda i: (0, 0)),
        ],
        out_specs=[
            pl.BlockSpec((BN, BF), lambda i: (i, 0)),
            pl.BlockSpec((BN, H * KA), lambda i: (i, 0)),
        ],
        out_shape=[
            jax.ShapeDtypeStruct((n, BF), jnp.float32),
            jax.ShapeDtypeStruct((n, H * KA), jnp.float32),
        ],
    )(x, Wb, Wc, bc.reshape(1, -1))


def _combine_body(s_ref, w_ref, q_ref, m_ref, cnt_ref, wt_ref, bias_ref,
                  g_ref, be_ref, out_ref):
    cnt = cnt_ref[...]
    inv = 1.0 / cnt
    mean = s_ref[...] * inv
    var = q_ref[...] * inv - mean * mean
    std = jnp.sqrt(jnp.maximum(var, 0.0) + 1e-5)
    aggs = (mean, w_ref[...], var, std, m_ref[...])
    cols = []
    for a in range(A):
        for b in range(B):
            cols.append(aggs[a][:, b * F:(b + 1) * F])
    aggflat = jnp.concatenate(cols, axis=1)

    kf = KA * F
    rowi = lax.broadcasted_iota(jnp.int32, (KA, kf), 0)
    coli = lax.broadcasted_iota(jnp.int32, (KA, kf), 1)
    E = (coli // F == rowi).astype(jnp.float32)
    ci = lax.broadcasted_iota(jnp.int32, (kf, F), 0)
    fi = lax.broadcasted_iota(jnp.int32, (kf, F), 1)
    S = (ci % F == fi).astype(jnp.float32)

    wt = wt_ref[...]
    outs = []
    for h in range(H):
        wt_h = wt[:, h * KA:(h + 1) * KA]
        wexp = jnp.dot(wt_h, E, preferred_element_type=jnp.float32, precision=lax.Precision.HIGHEST)
        z = jnp.dot(wexp * aggflat, S, preferred_element_type=jnp.float32, precision=lax.Precision.HIGHEST)
        outs.append(z)
    y = jnp.concatenate(outs, axis=1) + bias_ref[...]

    mu = jnp.mean(y, axis=-1, keepdims=True)
    v = jnp.mean((y - mu) ** 2, axis=-1, keepdims=True)
    y = (y - mu) / jnp.sqrt(v + 1e-5) * g_ref[...] + be_ref[...]
    out_ref[...] = jnp.maximum(y, 0.0)


def _combine(s, w, q, m, cnt, wt, bias, g, be):
    n = s.shape[0]
    grid = n // BN
    vec = lambda i: (i, 0)
    par = lambda i: (0, 0)
    return pl.pallas_call(
        _combine_body,
        grid=(grid,),
        in_specs=[
            pl.BlockSpec((BN, BF), vec),
            pl.BlockSpec((BN, BF), vec),
            pl.BlockSpec((BN, BF), vec),
            pl.BlockSpec((BN, BF), vec),
            pl.BlockSpec((BN, 1), vec),
            pl.BlockSpec((BN, H * KA), vec),
            pl.BlockSpec((1, HID), par),
            pl.BlockSpec((1, HID), par),
            pl.BlockSpec((1, HID), par),
        ],
        out_specs=pl.BlockSpec((BN, HID), vec),
        out_shape=jax.ShapeDtypeStruct((n, HID), jnp.float32),
    )(s, w, q, m, cnt, wt, bias.reshape(1, -1), g.reshape(1, -1),
      be.reshape(1, -1))


def _head_body(h_ref, w3_ref, b3_ref, g3_ref, be3_ref, w4_ref, b4_ref,
               out_ref):
    y = jnp.dot(h_ref[...], w3_ref[...], preferred_element_type=jnp.float32, precision=lax.Precision.HIGHEST)
    y = y + b3_ref[...]
    mu = jnp.mean(y, axis=-1, keepdims=True)
    v = jnp.mean((y - mu) ** 2, axis=-1, keepdims=True)
    y = (y - mu) / jnp.sqrt(v + 1e-5) * g3_ref[...] + be3_ref[...]
    y = jnp.maximum(y, 0.0)
    out_ref[...] = (
        jnp.dot(y, w4_ref[...], preferred_element_type=jnp.float32, precision=lax.Precision.HIGHEST)
        + b4_ref[...]
    )


def _head(h, W3, b3, g3, be3, W4, b4):
    n = h.shape[0]
    grid = n // BN
    d3 = W3.shape[1]
    d4 = W4.shape[1]
    par = lambda i: (0, 0)
    return pl.pallas_call(
        _head_body,
        grid=(grid,),
        in_specs=[
            pl.BlockSpec((BN, HID), lambda i: (i, 0)),
            pl.BlockSpec((HID, d3), par),
            pl.BlockSpec((1, d3), par),
            pl.BlockSpec((1, d3), par),
            pl.BlockSpec((1, d3), par),
            pl.BlockSpec((d3, d4), par),
            pl.BlockSpec((1, d4), par),
        ],
        out_specs=pl.BlockSpec((BN, d4), lambda i: (i, 0)),
        out_shape=jax.ShapeDtypeStruct((n, d4), jnp.float32),
    )(h, W3, b3.reshape(1, -1), g3.reshape(1, -1), be3.reshape(1, -1), W4,
      b4.reshape(1, -1))


def _aggregate(bases, src, dst, symw, n):
    msgs = bases[src]
    s = jnp.zeros((n, BF), jnp.float32).at[dst].add(msgs)
    w = jnp.zeros((n, BF), jnp.float32).at[dst].add(msgs * symw[:, None])
    q = jnp.zeros((n, BF), jnp.float32).at[dst].add(msgs * msgs)
    m = jnp.full((n, BF), -jnp.inf, jnp.float32).at[dst].max(msgs)
    m = jnp.where(jnp.isfinite(m), m, 0.0)
    return s, w, q, m


def kernel(x, edge_index, edge_weight, Wb1, Wc1, bc1, bias1, g1, be1,
           Wb2, Wc2, bc2, bias2, g2, be2, W3, b3, g3, be3, W4, b4):
    n = x.shape[0]
    loop = jnp.arange(n, dtype=edge_index.dtype)
    src = jnp.concatenate([edge_index[0], loop])
    dst = jnp.concatenate([edge_index[1], loop])

    deg = jnp.zeros((n,), jnp.float32).at[dst].add(1.0)
    dinv = jnp.where(deg > 0, lax.rsqrt(deg), 0.0)
    symw = dinv[src] * dinv[dst]
    cnt = jnp.maximum(deg, 1.0).reshape(n, 1)

    bases1, wt1 = _project(x, Wb1, Wc1, bc1)
    s, w, q, m = _aggregate(bases1, src, dst, symw, n)
    h1 = _combine(s, w, q, m, cnt, wt1, bias1, g1, be1)

    bases2, wt2 = _project(h1, Wb2, Wc2, bc2)
    s, w, q, m = _aggregate(bases2, src, dst, symw, n)
    h2 = _combine(s, w, q, m, cnt, wt2, bias2, g2, be2)

    return _head(h2, W3, b3, g3, be3, W4, b4)


# speedup vs baseline: 1.6890x; 1.5107x over previous
import functools

import jax
import jax.numpy as jnp
from jax import lax
from jax.experimental import pallas as pl
from jax.experimental.pallas import tpu as pltpu
from jax.experimental.pallas import tpu_sc as plsc

H, B, A = 8, 4, 5
F = 16
BF = B * F
BH = BF // 2
KA = A * B
HID = H * F
BN = 512
NP = 10240
NH = NP // 2
ETP = 180224
C = 352
NCH = ETP // 16 // C
RT = NH // 16
DUMP = NH
NEG = -3.0e38

_MESH = plsc.VectorSubcoreMesh(core_axis_name="c", subcore_axis_name="s")


def _project_body(x_ref, wb_ref, wc_ref, bc_ref, dinv_ref,
                  tab_ref, bt_ref, wt_ref):
    x = x_ref[...]
    bases = jnp.dot(x, wb_ref[...], preferred_element_type=jnp.float32,
                    precision=lax.Precision.HIGHEST)
    for c in range(2):
        b = bases[:, c * BH:(c + 1) * BH]
        bt_ref[c] = b
        tab_ref[c] = jnp.concatenate(
            [b, b * dinv_ref[...], b * b, jnp.zeros_like(b)], axis=1)
    wt_ref[...] = (
        jnp.dot(x, wc_ref[...], preferred_element_type=jnp.float32,
                precision=lax.Precision.HIGHEST)
        + bc_ref[...]
    )


def _project(x, Wb, Wc, bc, dinv):
    n, din = x.shape
    grid = n // BN
    return pl.pallas_call(
        _project_body,
        grid=(grid,),
        in_specs=[
            pl.BlockSpec((BN, din), lambda i: (i, 0)),
            pl.BlockSpec((din, BF), lambda i: (0, 0)),
            pl.BlockSpec((din, H * KA), lambda i: (0, 0)),
            pl.BlockSpec((1, H * KA), lambda i: (0, 0)),
            pl.BlockSpec((BN, 1), lambda i: (i, 0)),
        ],
        out_specs=[
            pl.BlockSpec((2, BN, 4 * BH), lambda i: (0, i, 0)),
            pl.BlockSpec((2, BN, BH), lambda i: (0, i, 0)),
            pl.BlockSpec((BN, H * KA), lambda i: (i, 0)),
        ],
        out_shape=[
            jax.ShapeDtypeStruct((2, n, 4 * BH), jnp.float32),
            jax.ShapeDtypeStruct((2, n, BH), jnp.float32),
            jax.ShapeDtypeStruct((n, H * KA), jnp.float32),
        ],
    )(x, Wb, Wc, bc.reshape(1, -1), dinv)


def _prep_body(degp_ref, cnt_ref, dinv_ref):
    deg = degp_ref[0]
    for k in range(1, 32):
        deg = deg + degp_ref[k]
    cnt_ref[...] = jnp.maximum(deg, 1.0)
    dinv_ref[...] = jnp.where(deg > 0, lax.rsqrt(deg), 0.0)


def _prep(degp):
    r = NP // 128
    return pl.pallas_call(
        _prep_body,
        in_specs=[pl.BlockSpec((32, r, 128), lambda: (0, 0, 0))],
        out_specs=[pl.BlockSpec((r, 128), lambda: (0, 0)),
                   pl.BlockSpec((r, 128), lambda: (0, 0))],
        out_shape=[jax.ShapeDtypeStruct((r, 128), jnp.float32),
                   jax.ShapeDtypeStruct((r, 128), jnp.float32)],
    )(degp.reshape(32, r, 128))


def _combine_body(cols, a_ref, b_ref, m_ref, cnt_ref, dinv_ref, wt_ref,
                  bias_ref, g_ref, be_ref, out_ref):
    s_col, w_col, q_col, w_from_b = cols

    def pick(ref, col):
        return jnp.concatenate(
            [ref[0][:, col:col + BH], ref[1][:, col:col + BH]], axis=1)

    inv = 1.0 / cnt_ref[...]
    mean = pick(a_ref, s_col) * inv
    var = pick(a_ref, q_col) * inv - mean * mean
    std = jnp.sqrt(jnp.maximum(var, 0.0) + 1e-5)
    symn = pick(b_ref if w_from_b else a_ref, w_col) * dinv_ref[...]
    mx = m_ref[0]
    for k in range(1, 8):
        mx = jnp.maximum(mx, m_ref[k])
    mx = jnp.where(mx < -1e38, 0.0, mx)
    aggs = (mean, symn, var, std, mx)
    cols = []
    for a in range(A):
        for b in range(B):
            cols.append(aggs[a][:, b * F:(b + 1) * F])
    aggflat = jnp.concatenate(cols, axis=1)

    kf = KA * F
    rowi = lax.broadcasted_iota(jnp.int32, (KA, kf), 0)
    coli = lax.broadcasted_iota(jnp.int32, (KA, kf), 1)
    E = (coli // F == rowi).astype(jnp.float32)
    ci = lax.broadcasted_iota(jnp.int32, (kf, F), 0)
    fi = lax.broadcasted_iota(jnp.int32, (kf, F), 1)
    S = (ci % F == fi).astype(jnp.float32)

    wt = wt_ref[...]
    outs = []
    for h in range(H):
        wt_h = wt[:, h * KA:(h + 1) * KA]
        wexp = jnp.dot(wt_h, E, preferred_element_type=jnp.float32,
                       precision=lax.Precision.HIGHEST)
        z = jnp.dot(wexp * aggflat, S, preferred_element_type=jnp.float32,
                    precision=lax.Precision.HIGHEST)
        outs.append(z)
    y = jnp.concatenate(outs, axis=1) + bias_ref[...]

    mu = jnp.mean(y, axis=-1, keepdims=True)
    v = jnp.mean((y - mu) ** 2, axis=-1, keepdims=True)
    y = (y - mu) / jnp.sqrt(v + 1e-5) * g_ref[...] + be_ref[...]
    out_ref[...] = jnp.maximum(y, 0.0)


def _combine(cols, a, b, m, cnt, dinv, wt, bias, g, be):
    n = a.shape[1]
    grid = n // BN
    wa = a.shape[2]
    wb = b.shape[2]
    vec = lambda i: (i, 0)
    par = lambda i: (0, 0)
    return pl.pallas_call(
        functools.partial(_combine_body, cols),
        grid=(grid,),
        in_specs=[
            pl.BlockSpec((2, BN, wa), lambda i: (0, i, 0)),
            pl.BlockSpec((2, BN, wb), lambda i: (0, i, 0)),
            pl.BlockSpec((8, BN, BF), lambda i: (0, i, 0)),
            pl.BlockSpec((BN, 1), vec),
            pl.BlockSpec((BN, 1), vec),
            pl.BlockSpec((BN, H * KA), vec),
            pl.BlockSpec((1, HID), par),
            pl.BlockSpec((1, HID), par),
            pl.BlockSpec((1, HID), par),
        ],
        out_specs=pl.BlockSpec((BN, HID), vec),
        out_shape=jax.ShapeDtypeStruct((n, HID), jnp.float32),
    )(a, b, m, cnt, dinv, wt, bias.reshape(1, -1), g.reshape(1, -1),
      be.reshape(1, -1))


def _head_body(h_ref, w3_ref, b3_ref, g3_ref, be3_ref, w4_ref, b4_ref,
               out_ref):
    y = jnp.dot(h_ref[...], w3_ref[...], preferred_element_type=jnp.float32,
                precision=lax.Precision.HIGHEST)
    y = y + b3_ref[...]
    mu = jnp.mean(y, axis=-1, keepdims=True)
    v = jnp.mean((y - mu) ** 2, axis=-1, keepdims=True)
    y = (y - mu) / jnp.sqrt(v + 1e-5) * g3_ref[...] + be3_ref[...]
    y = jnp.maximum(y, 0.0)
    out_ref[...] = (
        jnp.dot(y, w4_ref[...], preferred_element_type=jnp.float32,
                precision=lax.Precision.HIGHEST)
        + b4_ref[...]
    )


def _head(h, W3, b3, g3, be3, W4, b4):
    n = h.shape[0]
    grid = n // BN
    d3 = W3.shape[1]
    d4 = W4.shape[1]
    par = lambda i: (0, 0)
    return pl.pallas_call(
        _head_body,
        grid=(grid,),
        in_specs=[
            pl.BlockSpec((BN, HID), lambda i: (i, 0)),
            pl.BlockSpec((HID, d3), par),
            pl.BlockSpec((1, d3), par),
            pl.BlockSpec((1, d3), par),
            pl.BlockSpec((1, d3), par),
            pl.BlockSpec((d3, d4), par),
            pl.BlockSpec((1, d4), par),
        ],
        out_specs=pl.BlockSpec((BN, d4), lambda i: (i, 0)),
        out_shape=jax.ShapeDtypeStruct((n, d4), jnp.float32),
    )(h, W3, b3.reshape(1, -1), g3.reshape(1, -1), be3.reshape(1, -1), W4,
      b4.reshape(1, -1))


def _deg_body(dst_hbm, degp_hbm, dst_c, dacc, sem):
    c = lax.axis_index("c")
    s = lax.axis_index("s")
    wid = s * 2 + c
    lanes = lax.iota(jnp.int32, 16)

    def _fill0(i, _):
        dacc[pl.ds(i * 16, 16)] = jnp.zeros((16,), jnp.float32)
        return 0
    lax.fori_loop(0, NP // 16, _fill0, 0)

    woff = wid * (ETP // 32)

    def _chunk(t, _):
        pltpu.sync_copy(dst_hbm.at[pl.ds(woff + t * C, C)],
                        dst_c.at[pl.ds(0, C)])

        def _upd(e, _):
            d = dst_c[pl.ds(e, 16)][0]
            r = (d >> 4) << 4
            m = lanes == (d - r)
            v = dacc[pl.ds(r, 16)]
            dacc[pl.ds(r, 16)] = v + jnp.where(m, 1.0, 0.0)
            return 0
        lax.fori_loop(0, C, _upd, 0)
        return 0
    lax.fori_loop(0, ETP // 32 // C, _chunk, 0)
    pltpu.sync_copy(dacc, degp_hbm.at[wid])


def _deg(dstP):
    k = functools.partial(
        pl.kernel,
        out_type=jax.ShapeDtypeStruct((32, NP), jnp.float32),
        mesh=_MESH,
        scratch_types=[
            pltpu.VMEM((C + 16,), jnp.int32),
            pltpu.VMEM((NP,), jnp.float32),
            pltpu.SemaphoreType.DMA,
        ],
    )(_deg_body)
    return k(dstP)


def _agg_body(W, src_hbm, dst_hbm, tab_hbm, out,
              src_c, dst_c, srcadj, dstsel, rows, acc, sem):
    c = lax.axis_index("c")
    s = lax.axis_index("s")
    woff = s * (ETP // 16)
    tab_off = c * NP
    nz = W // 16

    for j in range(C // 16):
        srcadj[pl.ds(j * 16, 16)] = jnp.zeros((16,), jnp.int32)

    for p in range(2):
        lo = p * NH
        def _fill0(i, _):
            for j in range(nz):
                rows[i, pl.ds(j * 16, 16)] = jnp.zeros((16,), jnp.float32)
            return 0
        lax.fori_loop(0, C, _fill0, 0)
        pltpu.sync_copy(rows.at[pl.ds(0, RT)], acc.at[pl.ds(s * RT, RT)])
        @pl.when(s == 0)
        def _():
            pltpu.sync_copy(rows.at[pl.ds(0, NH + 16 - RT * 16)],
                            acc.at[pl.ds(RT * 16, NH + 16 - RT * 16)])
        plsc.subcore_barrier()

        def _chunk(t, _):
            base = woff + t * C
            pltpu.sync_copy(src_hbm.at[pl.ds(base, C)], src_c)
            pltpu.sync_copy(dst_hbm.at[pl.ds(base, C)], dst_c)

            def _sel(j, _):
                dv = dst_c[pl.ds(j * 16, 16)]
                iv = src_c[pl.ds(j * 16, 16)]
                m = (dv >= lo) & (dv < lo + NH)
                dstsel[pl.ds(j * 16, 16)] = jnp.where(
                    m, dv - lo, jnp.full((16,), DUMP, jnp.int32))
                srcadj[pl.ds(j * 16, 16)] = iv + tab_off
                return 0
            lax.fori_loop(0, C // 16, _sel, 0)

            pltpu.async_copy(tab_hbm.at[srcadj], rows, sem).wait()
            pltpu.sync_copy(rows, acc.at[dstsel], add=True)
            return 0
        lax.fori_loop(0, NCH, _chunk, 0)
        plsc.subcore_barrier()

        pltpu.sync_copy(acc.at[pl.ds(s * RT, RT)],
                        out.at[c, pl.ds(lo + s * RT, RT)])
        plsc.subcore_barrier()


def _agg(srcP, dstP, tab_flat, W):
    k = functools.partial(
        pl.kernel,
        out_type=jax.ShapeDtypeStruct((2, NP, W), jnp.float32),
        mesh=_MESH,
        scratch_types=[
            pltpu.VMEM((C,), jnp.int32),
            pltpu.VMEM((C,), jnp.int32),
            pltpu.VMEM((C,), jnp.int32),
            pltpu.VMEM((C,), jnp.int32),
            pltpu.VMEM((C, W), jnp.float32),
            pltpu.VMEM_SHARED((NH + 16, W), jnp.float32),
            pltpu.SemaphoreType.DMA,
        ],
    )(functools.partial(_agg_body, W))
    return k(srcP, dstP, tab_flat)


def _aggregate_max(bases, src, dst, n):
    msgs = bases[src]
    m = jnp.full((n, BF), NEG, jnp.float32).at[dst].max(msgs)
    return jnp.broadcast_to(m[None], (8, n, BF))


def kernel(x, edge_index, edge_weight, Wb1, Wc1, bc1, bias1, g1, be1,
           Wb2, Wc2, bc2, bias2, g2, be2, W3, b3, g3, be3, W4, b4):
    n = x.shape[0]
    loop = jnp.arange(n, dtype=edge_index.dtype)
    pad = ETP - (edge_index.shape[1] + n)
    srcP = jnp.concatenate(
        [edge_index[0], loop, jnp.zeros((pad,), edge_index.dtype)])
    dstP = jnp.concatenate(
        [edge_index[1], loop,
         jnp.full((pad,), NP - 1, edge_index.dtype)])
    xP = jnp.concatenate([x, jnp.zeros((NP - n, x.shape[1]), x.dtype)])

    degp = _deg(dstP)
    cnt, dinv = _prep(degp)
    cnt = cnt.reshape(NP, 1)
    dinv = dinv.reshape(NP, 1)

    tab1, bt1, wt1 = _project(xP, Wb1, Wc1, bc1, dinv)
    agga1 = _agg(srcP, dstP, tab1.reshape(2 * NP, 4 * BH), 4 * BH)
    m1 = _aggregate_max(jnp.concatenate([bt1[0], bt1[1]], axis=1),
                        srcP, dstP, NP)
    h1 = _combine((0, BH, 2 * BH, False), agga1, agga1, m1, cnt, dinv, wt1,
                  bias1, g1, be1)

    tab2, bt2, wt2 = _project(h1, Wb2, Wc2, bc2, dinv)
    agga2 = _agg(srcP, dstP, tab2.reshape(2 * NP, 4 * BH), 4 * BH)
    m2 = _aggregate_max(jnp.concatenate([bt2[0], bt2[1]], axis=1),
                        srcP, dstP, NP)
    h2 = _combine((0, BH, 2 * BH, False), agga2, agga2, m2, cnt, dinv, wt2,
                  bias2, g2, be2)

    return _head(h2, W3, b3, g3, be3, W4, b4)[:n]


# speedup vs baseline: 1.7294x; 1.0239x over previous
import functools

import jax
import jax.numpy as jnp
from jax import lax
from jax.experimental import pallas as pl
from jax.experimental.pallas import tpu as pltpu
from jax.experimental.pallas import tpu_sc as plsc

H, B, A = 8, 4, 5
F = 16
BF = B * F
BH = BF // 2
KA = A * B
HID = H * F
BN = 512
NP = 10240
NH = NP // 2
ETP = 180224
C = 352
NCH = ETP // 16 // C
RT = NH // 16
DUMP = NH
NEG = -3.0e38

_MESH = plsc.VectorSubcoreMesh(core_axis_name="c", subcore_axis_name="s")


def _project_body(x_ref, wb_ref, wc_ref, bc_ref, dinv_ref,
                  tab_ref, bt_ref, wt_ref):
    x = x_ref[...]
    bases = jnp.dot(x, wb_ref[...], preferred_element_type=jnp.float32,
                    precision=lax.Precision.HIGHEST)
    for c in range(2):
        b = bases[:, c * BH:(c + 1) * BH]
        bt_ref[c] = b
        tab_ref[c] = jnp.concatenate(
            [b, b * dinv_ref[...], b * b, jnp.zeros_like(b)], axis=1)
    wt_ref[...] = (
        jnp.dot(x, wc_ref[...], preferred_element_type=jnp.float32,
                precision=lax.Precision.HIGHEST)
        + bc_ref[...]
    )


def _project(x, Wb, Wc, bc, dinv):
    n, din = x.shape
    grid = n // BN
    return pl.pallas_call(
        _project_body,
        grid=(grid,),
        in_specs=[
            pl.BlockSpec((BN, din), lambda i: (i, 0)),
            pl.BlockSpec((din, BF), lambda i: (0, 0)),
            pl.BlockSpec((din, H * KA), lambda i: (0, 0)),
            pl.BlockSpec((1, H * KA), lambda i: (0, 0)),
            pl.BlockSpec((BN, 1), lambda i: (i, 0)),
        ],
        out_specs=[
            pl.BlockSpec((2, BN, 4 * BH), lambda i: (0, i, 0)),
            pl.BlockSpec((2, BN, BH), lambda i: (0, i, 0)),
            pl.BlockSpec((BN, H * KA), lambda i: (i, 0)),
        ],
        out_shape=[
            jax.ShapeDtypeStruct((2, n, 4 * BH), jnp.float32),
            jax.ShapeDtypeStruct((2, n, BH), jnp.float32),
            jax.ShapeDtypeStruct((n, H * KA), jnp.float32),
        ],
    )(x, Wb, Wc, bc.reshape(1, -1), dinv)


def _prep_body(degp_ref, cnt_ref, dinv_ref):
    deg = degp_ref[0]
    for k in range(1, 32):
        deg = deg + degp_ref[k]
    cnt_ref[...] = jnp.maximum(deg, 1.0)
    dinv_ref[...] = jnp.where(deg > 0, lax.rsqrt(deg), 0.0)


def _prep(degp):
    r = NP // 128
    return pl.pallas_call(
        _prep_body,
        in_specs=[pl.BlockSpec((32, r, 128), lambda: (0, 0, 0))],
        out_specs=[pl.BlockSpec((r, 128), lambda: (0, 0)),
                   pl.BlockSpec((r, 128), lambda: (0, 0))],
        out_shape=[jax.ShapeDtypeStruct((r, 128), jnp.float32),
                   jax.ShapeDtypeStruct((r, 128), jnp.float32)],
    )(degp.reshape(32, r, 128))


def _combine_body(cols, a_ref, b_ref, m_ref, cnt_ref, dinv_ref, wt_ref,
                  bias_ref, g_ref, be_ref, out_ref):
    s_col, w_col, q_col, w_from_b = cols

    def pick(ref, col):
        return jnp.concatenate(
            [ref[0][:, col:col + BH], ref[1][:, col:col + BH]], axis=1)

    inv = 1.0 / cnt_ref[...]
    mean = pick(a_ref, s_col) * inv
    var = pick(a_ref, q_col) * inv - mean * mean
    std = jnp.sqrt(jnp.maximum(var, 0.0) + 1e-5)
    symn = pick(b_ref if w_from_b else a_ref, w_col) * dinv_ref[...]
    mx = m_ref[0]
    for k in range(1, m_ref.shape[0]):
        mx = jnp.maximum(mx, m_ref[k])
    mx = jnp.where(mx < -1e38, 0.0, mx)
    aggs = (mean, symn, var, std, mx)
    cols = []
    for a in range(A):
        for b in range(B):
            cols.append(aggs[a][:, b * F:(b + 1) * F])
    aggflat = jnp.concatenate(cols, axis=1)

    kf = KA * F
    rowi = lax.broadcasted_iota(jnp.int32, (KA, kf), 0)
    coli = lax.broadcasted_iota(jnp.int32, (KA, kf), 1)
    E = (coli // F == rowi).astype(jnp.float32)
    ci = lax.broadcasted_iota(jnp.int32, (kf, F), 0)
    fi = lax.broadcasted_iota(jnp.int32, (kf, F), 1)
    S = (ci % F == fi).astype(jnp.float32)

    wt = wt_ref[...]
    outs = []
    for h in range(H):
        wt_h = wt[:, h * KA:(h + 1) * KA]
        wexp = jnp.dot(wt_h, E, preferred_element_type=jnp.float32,
                       precision=lax.Precision.HIGHEST)
        z = jnp.dot(wexp * aggflat, S, preferred_element_type=jnp.float32,
                    precision=lax.Precision.HIGHEST)
        outs.append(z)
    y = jnp.concatenate(outs, axis=1) + bias_ref[...]

    mu = jnp.mean(y, axis=-1, keepdims=True)
    v = jnp.mean((y - mu) ** 2, axis=-1, keepdims=True)
    y = (y - mu) / jnp.sqrt(v + 1e-5) * g_ref[...] + be_ref[...]
    out_ref[...] = jnp.maximum(y, 0.0)


def _combine(cols, a, b, m, cnt, dinv, wt, bias, g, be):
    n = a.shape[1]
    grid = n // BN
    wa = a.shape[2]
    wb = b.shape[2]
    vec = lambda i: (i, 0)
    par = lambda i: (0, 0)
    return pl.pallas_call(
        functools.partial(_combine_body, cols),
        grid=(grid,),
        in_specs=[
            pl.BlockSpec((2, BN, wa), lambda i: (0, i, 0)),
            pl.BlockSpec((2, BN, wb), lambda i: (0, i, 0)),
            pl.BlockSpec((m.shape[0], BN, BF), lambda i: (0, i, 0)),
            pl.BlockSpec((BN, 1), vec),
            pl.BlockSpec((BN, 1), vec),
            pl.BlockSpec((BN, H * KA), vec),
            pl.BlockSpec((1, HID), par),
            pl.BlockSpec((1, HID), par),
            pl.BlockSpec((1, HID), par),
        ],
        out_specs=pl.BlockSpec((BN, HID), vec),
        out_shape=jax.ShapeDtypeStruct((n, HID), jnp.float32),
    )(a, b, m, cnt, dinv, wt, bias.reshape(1, -1), g.reshape(1, -1),
      be.reshape(1, -1))


def _head_body(h_ref, w3_ref, b3_ref, g3_ref, be3_ref, w4_ref, b4_ref,
               out_ref):
    y = jnp.dot(h_ref[...], w3_ref[...], preferred_element_type=jnp.float32,
                precision=lax.Precision.HIGHEST)
    y = y + b3_ref[...]
    mu = jnp.mean(y, axis=-1, keepdims=True)
    v = jnp.mean((y - mu) ** 2, axis=-1, keepdims=True)
    y = (y - mu) / jnp.sqrt(v + 1e-5) * g3_ref[...] + be3_ref[...]
    y = jnp.maximum(y, 0.0)
    out_ref[...] = (
        jnp.dot(y, w4_ref[...], preferred_element_type=jnp.float32,
                precision=lax.Precision.HIGHEST)
        + b4_ref[...]
    )


def _head(h, W3, b3, g3, be3, W4, b4):
    n = h.shape[0]
    grid = n // BN
    d3 = W3.shape[1]
    d4 = W4.shape[1]
    par = lambda i: (0, 0)
    return pl.pallas_call(
        _head_body,
        grid=(grid,),
        in_specs=[
            pl.BlockSpec((BN, HID), lambda i: (i, 0)),
            pl.BlockSpec((HID, d3), par),
            pl.BlockSpec((1, d3), par),
            pl.BlockSpec((1, d3), par),
            pl.BlockSpec((1, d3), par),
            pl.BlockSpec((d3, d4), par),
            pl.BlockSpec((1, d4), par),
        ],
        out_specs=pl.BlockSpec((BN, d4), lambda i: (i, 0)),
        out_shape=jax.ShapeDtypeStruct((n, d4), jnp.float32),
    )(h, W3, b3.reshape(1, -1), g3.reshape(1, -1), be3.reshape(1, -1), W4,
      b4.reshape(1, -1))


def _deg_body(dst_hbm, degp_hbm, dst_c, dacc, sem):
    c = lax.axis_index("c")
    s = lax.axis_index("s")
    wid = s * 2 + c
    lanes = lax.iota(jnp.int32, 16)

    def _fill0(i, _):
        dacc[pl.ds(i * 16, 16)] = jnp.zeros((16,), jnp.float32)
        return 0
    lax.fori_loop(0, NP // 16, _fill0, 0)

    woff = wid * (ETP // 32)

    def _chunk(t, _):
        pltpu.sync_copy(dst_hbm.at[pl.ds(woff + t * C, C)],
                        dst_c.at[pl.ds(0, C)])

        def _upd(e, _):
            d = dst_c[pl.ds(e, 16)][0]
            r = (d >> 4) << 4
            m = lanes == (d - r)
            v = dacc[pl.ds(r, 16)]
            dacc[pl.ds(r, 16)] = v + jnp.where(m, 1.0, 0.0)
            return 0
        lax.fori_loop(0, C, _upd, 0)
        return 0
    lax.fori_loop(0, ETP // 32 // C, _chunk, 0)
    pltpu.sync_copy(dacc, degp_hbm.at[wid])


def _deg(dstP):
    k = functools.partial(
        pl.kernel,
        out_type=jax.ShapeDtypeStruct((32, NP), jnp.float32),
        mesh=_MESH,
        scratch_types=[
            pltpu.VMEM((C + 16,), jnp.int32),
            pltpu.VMEM((NP,), jnp.float32),
            pltpu.SemaphoreType.DMA,
        ],
    )(_deg_body)
    return k(dstP)


def _agg_body(W, src_hbm, dst_hbm, tab_hbm, out,
              src_c, dst_c, srcadj, dstsel, rows, acc, sem):
    c = lax.axis_index("c")
    s = lax.axis_index("s")
    woff = s * (ETP // 16)
    tab_off = c * NP
    nz = W // 16

    for j in range(C // 16):
        srcadj[pl.ds(j * 16, 16)] = jnp.zeros((16,), jnp.int32)

    for p in range(2):
        lo = p * NH
        def _fill0(i, _):
            for j in range(nz):
                rows[i, pl.ds(j * 16, 16)] = jnp.zeros((16,), jnp.float32)
            return 0
        lax.fori_loop(0, C, _fill0, 0)
        pltpu.sync_copy(rows.at[pl.ds(0, RT)], acc.at[pl.ds(s * RT, RT)])
        @pl.when(s == 0)
        def _():
            pltpu.sync_copy(rows.at[pl.ds(0, NH + 16 - RT * 16)],
                            acc.at[pl.ds(RT * 16, NH + 16 - RT * 16)])
        plsc.subcore_barrier()

        def _chunk(t, _):
            base = woff + t * C
            pltpu.sync_copy(src_hbm.at[pl.ds(base, C)], src_c)
            pltpu.sync_copy(dst_hbm.at[pl.ds(base, C)], dst_c)

            def _sel(j, _):
                dv = dst_c[pl.ds(j * 16, 16)]
                iv = src_c[pl.ds(j * 16, 16)]
                m = (dv >= lo) & (dv < lo + NH)
                dstsel[pl.ds(j * 16, 16)] = jnp.where(
                    m, dv - lo, jnp.full((16,), DUMP, jnp.int32))
                srcadj[pl.ds(j * 16, 16)] = iv + tab_off
                return 0
            lax.fori_loop(0, C // 16, _sel, 0)

            pltpu.async_copy(tab_hbm.at[srcadj], rows, sem).wait()
            pltpu.sync_copy(rows, acc.at[dstsel], add=True)
            return 0
        lax.fori_loop(0, NCH, _chunk, 0)
        plsc.subcore_barrier()

        pltpu.sync_copy(acc.at[pl.ds(s * RT, RT)],
                        out.at[c, pl.ds(lo + s * RT, RT)])
        plsc.subcore_barrier()


def _agg(srcP, dstP, tab_flat, W):
    k = functools.partial(
        pl.kernel,
        out_type=jax.ShapeDtypeStruct((2, NP, W), jnp.float32),
        mesh=_MESH,
        scratch_types=[
            pltpu.VMEM((C,), jnp.int32),
            pltpu.VMEM((C,), jnp.int32),
            pltpu.VMEM((C,), jnp.int32),
            pltpu.VMEM((C,), jnp.int32),
            pltpu.VMEM((C, W), jnp.float32),
            pltpu.VMEM_SHARED((NH + 16, W), jnp.float32),
            pltpu.SemaphoreType.DMA,
        ],
    )(functools.partial(_agg_body, W))
    return k(srcP, dstP, tab_flat)


def _aggregate_max(bases, src, dst, n):
    msgs = bases[src]
    m = jnp.full((n, BF), NEG, jnp.float32).at[dst].max(msgs)
    return m[None]


def kernel(x, edge_index, edge_weight, Wb1, Wc1, bc1, bias1, g1, be1,
           Wb2, Wc2, bc2, bias2, g2, be2, W3, b3, g3, be3, W4, b4):
    n = x.shape[0]
    loop = jnp.arange(n, dtype=edge_index.dtype)
    pad = ETP - (edge_index.shape[1] + n)
    srcP = jnp.concatenate(
        [edge_index[0], loop, jnp.zeros((pad,), edge_index.dtype)])
    dstP = jnp.concatenate(
        [edge_index[1], loop,
         jnp.full((pad,), NP - 1, edge_index.dtype)])
    xP = jnp.concatenate([x, jnp.zeros((NP - n, x.shape[1]), x.dtype)])

    degp = _deg(dstP)
    cnt, dinv = _prep(degp)
    cnt = cnt.reshape(NP, 1)
    dinv = dinv.reshape(NP, 1)

    tab1, bt1, wt1 = _project(xP, Wb1, Wc1, bc1, dinv)
    agga1 = _agg(srcP, dstP, tab1.reshape(2 * NP, 4 * BH), 4 * BH)
    m1 = _aggregate_max(jnp.concatenate([bt1[0], bt1[1]], axis=1),
                        srcP, dstP, NP)
    h1 = _combine((0, BH, 2 * BH, False), agga1, agga1, m1, cnt, dinv, wt1,
                  bias1, g1, be1)

    tab2, bt2, wt2 = _project(h1, Wb2, Wc2, bc2, dinv)
    agga2 = _agg(srcP, dstP, tab2.reshape(2 * NP, 4 * BH), 4 * BH)
    m2 = _aggregate_max(jnp.concatenate([bt2[0], bt2[1]], axis=1),
                        srcP, dstP, NP)
    h2 = _combine((0, BH, 2 * BH, False), agga2, agga2, m2, cnt, dinv, wt2,
                  bias2, g2, be2)

    return _head(h2, W3, b3, g3, be3, W4, b4)[:n]
